# baseline (speedup 1.0000x reference)
import functools

import jax
import jax.numpy as jnp
import numpy as np
from jax import lax
from jax.experimental import pallas as pl
from jax.experimental.pallas import tpu as pltpu
from jax.experimental.pallas import tpu_sc as plsc

D = 128
NC = 2
NS = 16
NW = NC * NS
CH = 128
LANES = 16
NB = 4
PF = 3


def _mesh():
    return plsc.VectorSubcoreMesh(
        core_axis_name="c", subcore_axis_name="s", num_cores=NC, num_subcores=NS
    )


@functools.cache
def _deg_call(n_pad: int, kd: int):
    rows = n_pad // NS

    @functools.partial(
        pl.kernel,
        out_type=jax.ShapeDtypeStruct((NC, n_pad), jnp.float32),
        mesh=_mesh(),
        compiler_params=pltpu.CompilerParams(use_tc_tiling_on_sc=False),
        scratch_types=[
            pltpu.VMEM((kd, CH), jnp.int32),
            pltpu.VMEM((CH,), jnp.float32),
            pltpu.VMEM((rows,), jnp.float32),
            pltpu.VMEM_SHARED((n_pad,), jnp.float32),
        ],
    )
    def deg(idx_hbm, out_hbm, idx_v, ones_v, z_v, acc):
        c = lax.axis_index("c")
        s = lax.axis_index("s")
        pltpu.sync_copy(idx_hbm.at[c, s], idx_v)
        for k in range(CH // LANES):
            ones_v[pl.ds(k * LANES, LANES)] = jnp.ones((LANES,), jnp.float32)

        def zbody(r, carry):
            z_v[pl.ds(r * LANES, LANES)] = jnp.zeros((LANES,), jnp.float32)
            return carry

        lax.fori_loop(0, rows // LANES, zbody, 0)
        pltpu.sync_copy(z_v, acc.at[pl.ds(s * rows, rows)])
        plsc.subcore_barrier()

        def body(j, carry):
            pltpu.sync_copy(ones_v, acc.at[idx_v.at[j]], add=True)
            return carry

        lax.fori_loop(0, kd, body, 0)
        plsc.subcore_barrier()
        pltpu.sync_copy(
            acc.at[pl.ds(s * rows, rows)],
            out_hbm.at[c, pl.ds(s * rows, rows)],
        )

    return deg


@functools.cache
def _agg_call(n_pad: int, kc: int):
    rows = n_pad // NS
    hd = D // NC
    assert rows % CH == 0 and kc % NB == 0 and PF < NB
    himask = jnp.int32(-65536)

    @functools.partial(
        pl.kernel,
        out_type=jax.ShapeDtypeStruct((NC, n_pad, hd), jnp.float32),
        mesh=_mesh(),
        compiler_params=pltpu.CompilerParams(
            use_tc_tiling_on_sc=False, needs_layout_passes=False
        ),
        scratch_types=[
            pltpu.VMEM((kc, CH), jnp.int32),
            pltpu.VMEM((kc, CH), jnp.int32),
            [pltpu.VMEM((CH, hd // 2), jnp.int32) for _ in range(NB)],
            [pltpu.VMEM((CH, hd), jnp.float32) for _ in range(2)],
            pltpu.VMEM_SHARED((n_pad, hd), jnp.float32),
            [pltpu.SemaphoreType.DMA for _ in range(NB)],
            [pltpu.SemaphoreType.DMA for _ in range(2)],
        ],
    )
    def agg(hi_hbm, src_hbm, dst_hbm, out_hbm, src_v, dst_v, gb, fb, acc, gsems, ssems):
        c = lax.axis_index("c")
        s = lax.axis_index("s")
        pltpu.sync_copy(src_hbm.at[c, s], src_v)
        pltpu.sync_copy(dst_hbm.at[s], dst_v)

        def zbody(r, carry):
            for k in range(hd // LANES):
                fb[0][r, pl.ds(k * LANES, LANES)] = jnp.zeros((LANES,), jnp.float32)
            return carry

        lax.fori_loop(0, CH, zbody, 0)
        for blk in range(rows // CH):
            pltpu.sync_copy(fb[0], acc.at[pl.ds(s * rows + blk * CH, CH)])
        plsc.subcore_barrier()

        def widen(b, p):
            def wbody(rr, carry):
                for ri in range(8):
                    r = rr * 8 + ri
                    for k in range(hd // 32):
                        v = gb[b][r, pl.ds(k * LANES, LANES)]
                        odd = plsc.bitcast(v & himask, jnp.float32)
                        even = plsc.bitcast(v << 16, jnp.float32)
                        fb[p][r, pl.ds(32 * k, LANES)] = odd
                        fb[p][r, pl.ds(32 * k + LANES, LANES)] = even
                return carry

            lax.fori_loop(0, CH // 8, wbody, 0)

        for b in range(PF):
            pltpu.async_copy(hi_hbm.at[src_v.at[b]], gb[b], gsems[b])

        def body(i, carry):
            base = i * NB
            for b in range(NB):
                j = base + b
                p = b % 2
                pltpu.make_async_copy(hi_hbm.at[src_v.at[j]], gb[b], gsems[b]).wait()
                jp = j + PF
                bp = (b + PF) % NB

                @pl.when(jp < kc)
                def _():
                    pltpu.async_copy(hi_hbm.at[src_v.at[jp]], gb[bp], gsems[bp])

                widen(b, p)

                @pl.when(j >= 1)
                def _():
                    pltpu.make_async_copy(
                        fb[1 - p], acc.at[dst_v.at[0]], ssems[1 - p]
                    ).wait()

                pltpu.async_copy(fb[p], acc.at[dst_v.at[j]], ssems[p], add=True)

            return carry

        lax.fori_loop(0, kc // NB, body, 0)
        pltpu.make_async_copy(fb[(kc - 1) % 2], acc.at[dst_v.at[0]], ssems[(kc - 1) % 2]).wait()
        plsc.subcore_barrier()
        pltpu.sync_copy(
            acc.at[pl.ds(s * rows, rows)],
            out_hbm.at[c, pl.ds(s * rows, rows)],
        )

    return agg


def _prep_tc(x_pad, deg_t):
    def body(x_ref, deg_ref, h_ref):
        norm = lax.rsqrt(jnp.maximum(deg_ref[:, 0:1], 1.0))
        h_ref[...] = (x_ref[...] * norm).astype(jnp.bfloat16)

    return pl.pallas_call(
        body,
        out_shape=jax.ShapeDtypeStruct(x_pad.shape, jnp.bfloat16),
    )(x_pad, deg_t)


def _finish_tc(partials, deg_t, w, b2):
    n_pad = partials.shape[1]

    def body(p_ref, deg_ref, w_ref, b_ref, o_ref):
        p = jnp.concatenate([p_ref[0], p_ref[1]], axis=1)
        norm = lax.rsqrt(jnp.maximum(deg_ref[:, 1:2], 1.0))
        agg = p * norm
        o_ref[...] = (
            jnp.dot(agg, w_ref[...], preferred_element_type=jnp.float32) + b_ref[...]
        )

    return pl.pallas_call(
        body,
        out_shape=jax.ShapeDtypeStruct((n_pad, D), jnp.float32),
    )(partials, deg_t, w, b2)


def kernel(x, edge_index, W, b):
    n, d = x.shape
    assert d == D
    e = edge_index.shape[1]

    n_pad = -(-n // (NS * CH)) * (NS * CH)
    if n_pad == n:
        n_pad += NS * CH
    e_per_t = -(-e // (NS * NB * CH)) * (NB * CH)
    e_pad = e_per_t * NS
    kc = e_per_t // CH

    src = edge_index[0]
    dst = edge_index[1]
    spare = n_pad - n
    fill = (jnp.arange(e_pad - e, dtype=jnp.int32) % spare) + n
    src_p = jnp.concatenate([src, fill])
    dst_p = jnp.concatenate([dst, fill])

    idx2 = jnp.stack([src_p, dst_p]).reshape(NC, NS, kc, CH)
    counts = _deg_call(n_pad, kc)(idx2)
    deg_t = counts.T

    x_pad = jnp.pad(x, ((0, n_pad - n), (0, 0)))
    h = _prep_tc(x_pad, deg_t)
    hd = D // NC
    hi = jax.lax.bitcast_convert_type(
        h.reshape(2 * n_pad, hd // 2, 2), jnp.int32
    )

    srcg = jnp.stack([2 * src_p, 2 * src_p + 1]).reshape(NC, NS, kc, CH)
    partials = _agg_call(n_pad, kc)(hi, srcg, dst_p.reshape(NS, kc, CH))
    base = np.arange(32)
    perm_base = np.where(base % 2 == 1, base // 2, 16 + base // 2)
    perm = jnp.asarray(
        np.concatenate([g * 32 + perm_base for g in range(hd // 32)]),
        dtype=jnp.int32,
    )
    partials = partials[:, :, perm]
    out = _finish_tc(partials, deg_t, W, b.reshape(1, D))
    return out[:n]


# speedup vs baseline: 4.8177x; 4.8177x over previous
import functools

import jax
import jax.numpy as jnp
from jax import lax
from jax.experimental import pallas as pl
from jax.experimental.pallas import tpu as pltpu
from jax.experimental.pallas import tpu_sc as plsc

D = 128
NC = 2
NS = 16
CH = 128
LANES = 16
NB = 4
PF = 3
MAGIC = 0x5F3759DF


def _mesh():
    return plsc.VectorSubcoreMesh(
        core_axis_name="c", subcore_axis_name="s", num_cores=NC, num_subcores=NS
    )


@functools.cache
def _gcn_call(n_pad: int, kc: int):
    rows = n_pad // NS
    hd = D // NC
    assert rows % CH == 0 and kc % NB == 0 and kc % 2 == 0 and PF < NB

    @functools.partial(
        pl.kernel,
        out_type=[
            jax.ShapeDtypeStruct((NC, n_pad, hd), jnp.float32),
            jax.ShapeDtypeStruct((NC * n_pad, hd), jnp.float32),
            jax.ShapeDtypeStruct((NC, n_pad), jnp.float32),
        ],
        mesh=_mesh(),
        compiler_params=pltpu.CompilerParams(
            use_tc_tiling_on_sc=False, needs_layout_passes=False
        ),
        scratch_types=[
            pltpu.VMEM((kc, CH), jnp.int32),
            pltpu.VMEM((kc, CH), jnp.int32),
            pltpu.VMEM((CH,), jnp.float32),
            pltpu.VMEM((rows,), jnp.float32),
            pltpu.VMEM((2 * rows,), jnp.float32),
            [pltpu.VMEM((CH, hd), jnp.float32) for _ in range(NB)],
            pltpu.VMEM_SHARED((n_pad, hd), jnp.float32),
            pltpu.VMEM_SHARED((NC * n_pad,), jnp.float32),
            pltpu.VMEM_SHARED((n_pad,), jnp.float32),
            [pltpu.SemaphoreType.DMA for _ in range(NB)],
            [pltpu.SemaphoreType.DMA for _ in range(NB)],
        ],
    )
    def gcn(
        xs_hbm, src_hbm, dst_hbm,
        out_hbm, h2_hbm, cntd_hbm,
        src_v, dst_v, ones_v, nrm_v, z_v, bufs, acc, cnts, cntd, gsems, ssems,
    ):
        c = lax.axis_index("c")
        s = lax.axis_index("s")

        pltpu.sync_copy(src_hbm.at[c, s], src_v)
        pltpu.sync_copy(dst_hbm.at[s], dst_v)
        for k in range(CH // LANES):
            ones_v[pl.ds(k * LANES, LANES)] = jnp.ones((LANES,), jnp.float32)

        def zvec(r, carry):
            z_v[pl.ds(r * LANES, LANES)] = jnp.zeros((LANES,), jnp.float32)
            return carry

        lax.fori_loop(0, 2 * rows // LANES, zvec, 0)

        def zbuf(r, carry):
            for k in range(hd // LANES):
                bufs[0][r, pl.ds(k * LANES, LANES)] = jnp.zeros((LANES,), jnp.float32)
            return carry

        lax.fori_loop(0, CH, zbuf, 0)
        for blk in range(rows // CH):
            pltpu.sync_copy(bufs[0], acc.at[pl.ds(s * rows + blk * CH, CH)])
        pltpu.sync_copy(z_v, cnts.at[pl.ds(s * 2 * rows, 2 * rows)])
        pltpu.sync_copy(z_v.at[pl.ds(0, rows)], cntd.at[pl.ds(s * rows, rows)])
        plsc.subcore_barrier()

        def cbody(j, carry):
            pltpu.sync_copy(ones_v, cnts.at[src_v.at[j]], add=True)
            return carry

        lax.fori_loop(0, kc, cbody, 0)

        def dbody(j, carry):
            pltpu.sync_copy(ones_v, cntd.at[dst_v.at[c * (kc // 2) + j]], add=True)
            return carry

        lax.fori_loop(0, kc // 2, dbody, 0)
        plsc.subcore_barrier()

        pltpu.sync_copy(cnts.at[pl.ds(c * n_pad + s * rows, rows)], nrm_v)
        for k in range(rows // LANES):
            v = jnp.maximum(nrm_v[pl.ds(k * LANES, LANES)], 1.0)
            y = plsc.bitcast(MAGIC - (plsc.bitcast(v, jnp.int32) >> 1), jnp.float32)
            for _ in range(3):
                y = y * (1.5 - 0.5 * v * y * y)
            nrm_v[pl.ds(k * LANES, LANES)] = y

        def sblk(blk, carry):
            pltpu.sync_copy(xs_hbm.at[c, pl.ds(s * rows + blk * CH, CH)], bufs[0])

            def srow(g, carry2):
                nv = nrm_v[pl.ds(blk * CH + g * LANES, LANES)]
                for ri in range(LANES):
                    scl = nv[ri]
                    r = g * LANES + ri
                    for k in range(hd // LANES):
                        bufs[0][r, pl.ds(k * LANES, LANES)] = (
                            bufs[0][r, pl.ds(k * LANES, LANES)] * scl
                        )
                return carry2

            lax.fori_loop(0, CH // LANES, srow, 0)
            pltpu.sync_copy(
                bufs[0], h2_hbm.at[pl.ds(c * n_pad + s * rows + blk * CH, CH)]
            )
            return carry

        lax.fori_loop(0, rows // CH, sblk, 0)
        plsc.subcore_barrier()

        for b in range(PF):
            pltpu.async_copy(h2_hbm.at[src_v.at[b]], bufs[b], gsems[b])

        def body(i, carry):
            base = i * NB
            for b in range(NB):
                j = base + b
                pltpu.make_async_copy(h2_hbm.at[src_v.at[j]], bufs[b], gsems[b]).wait()
                bprev = (b - 1) % NB
                if b == 0:
                    @pl.when(i > 0)
                    def _():
                        pltpu.make_async_copy(
                            bufs[bprev], acc.at[dst_v.at[0]], ssems[bprev]
                        ).wait()
                else:
                    pltpu.make_async_copy(
                        bufs[bprev], acc.at[dst_v.at[0]], ssems[bprev]
                    ).wait()
                pltpu.async_copy(bufs[b], acc.at[dst_v.at[j]], ssems[b], add=True)
                jp = j + PF
                bp = (b + PF) % NB

                @pl.when(jp < kc)
                def _():
                    pltpu.async_copy(h2_hbm.at[src_v.at[jp]], bufs[bp], gsems[bp])

            return carry

        lax.fori_loop(0, kc // NB, body, 0)
        pltpu.make_async_copy(bufs[NB - 1], acc.at[dst_v.at[0]], ssems[NB - 1]).wait()
        plsc.subcore_barrier()

        pltpu.sync_copy(
            acc.at[pl.ds(s * rows, rows)],
            out_hbm.at[c, pl.ds(s * rows, rows)],
        )
        pltpu.sync_copy(
            cntd.at[pl.ds(s * rows, rows)],
            cntd_hbm.at[c, pl.ds(s * rows, rows)],
        )

    return gcn


def _finish_tc(partials, cntd_t, w, b2):
    n_pad = partials.shape[1]

    def body(p_ref, cd_ref, w_ref, b_ref, o_ref):
        p = jnp.concatenate([p_ref[0], p_ref[1]], axis=1)
        deg_in = cd_ref[:, 0:1] + cd_ref[:, 1:2]
        norm = lax.rsqrt(jnp.maximum(deg_in, 1.0))
        agg = p * norm
        o_ref[...] = (
            jnp.dot(agg, w_ref[...], preferred_element_type=jnp.float32) + b_ref[...]
        )

    return pl.pallas_call(
        body,
        out_shape=jax.ShapeDtypeStruct((n_pad, D), jnp.float32),
    )(partials, cntd_t, w, b2)


def kernel(x, edge_index, W, b):
    n, d = x.shape
    assert d == D
    e = edge_index.shape[1]

    n_pad = -(-n // (NS * CH)) * (NS * CH)
    if n_pad == n:
        n_pad += NS * CH
    e_per_t = -(-e // (NS * NB * CH)) * (NB * CH)
    e_pad = e_per_t * NS
    kc = e_per_t // CH

    src = edge_index[0]
    dst = edge_index[1]
    spare = n_pad - n
    fill = (jnp.arange(e_pad - e, dtype=jnp.int32) % spare) + n
    src_p = jnp.concatenate([src, fill])
    dst_p = jnp.concatenate([dst, fill])

    hd = D // NC
    x_pad = jnp.pad(x, ((0, n_pad - n), (0, 0)))
    xs = x_pad.reshape(n_pad, NC, hd).transpose(1, 0, 2)
    srcg = jnp.stack([src_p, src_p + n_pad]).reshape(NC, NS, kc, CH)
    partials, _, cntd = _gcn_call(n_pad, kc)(
        xs, srcg, dst_p.reshape(NS, kc, CH)
    )
    out = _finish_tc(partials, cntd.T, W, b.reshape(1, D))
    return out[:n]


# speedup vs baseline: 5.7827x; 1.2003x over previous
import functools

import jax
import jax.numpy as jnp
from jax import lax
from jax.experimental import pallas as pl
from jax.experimental.pallas import tpu as pltpu
from jax.experimental.pallas import tpu_sc as plsc

D = 128
NC = 2
NS = 16
NW = NC * NS
CH = 128
LANES = 16
NB = 4
PF = 3


def _mesh():
    return plsc.VectorSubcoreMesh(
        core_axis_name="c", subcore_axis_name="s", num_cores=NC, num_subcores=NS
    )


@functools.cache
def _deg_call(n_pad: int, kd: int):
    rows = n_pad // NS

    @functools.partial(
        pl.kernel,
        out_type=jax.ShapeDtypeStruct((NC, n_pad), jnp.float32),
        mesh=_mesh(),
        compiler_params=pltpu.CompilerParams(use_tc_tiling_on_sc=False),
        scratch_types=[
            pltpu.VMEM((kd, CH), jnp.int32),
            pltpu.VMEM((CH,), jnp.float32),
            pltpu.VMEM((rows,), jnp.float32),
            pltpu.VMEM_SHARED((n_pad,), jnp.float32),
        ],
    )
    def deg(idx_hbm, out_hbm, idx_v, ones_v, z_v, acc):
        c = lax.axis_index("c")
        s = lax.axis_index("s")
        pltpu.sync_copy(idx_hbm.at[c, s], idx_v)
        for k in range(CH // LANES):
            ones_v[pl.ds(k * LANES, LANES)] = jnp.ones((LANES,), jnp.float32)

        def zbody(r, carry):
            z_v[pl.ds(r * LANES, LANES)] = jnp.zeros((LANES,), jnp.float32)
            return carry

        lax.fori_loop(0, rows // LANES, zbody, 0)
        pltpu.sync_copy(z_v, acc.at[pl.ds(s * rows, rows)])
        plsc.subcore_barrier()

        def body(j, carry):
            pltpu.sync_copy(ones_v, acc.at[idx_v.at[j]], add=True)
            return carry

        lax.fori_loop(0, kd, body, 0)
        plsc.subcore_barrier()
        pltpu.sync_copy(
            acc.at[pl.ds(s * rows, rows)],
            out_hbm.at[c, pl.ds(s * rows, rows)],
        )

    return deg


@functools.cache
def _agg_call(n_pad: int, kc: int):
    rows = n_pad // NS
    hd = D // NC
    assert rows % CH == 0 and kc % NB == 0 and PF < NB

    @functools.partial(
        pl.kernel,
        out_type=jax.ShapeDtypeStruct((NC, n_pad, hd), jnp.float32),
        mesh=_mesh(),
        compiler_params=pltpu.CompilerParams(use_tc_tiling_on_sc=False),
        scratch_types=[
            pltpu.VMEM((kc, CH), jnp.int32),
            pltpu.VMEM((kc, CH), jnp.int32),
            [pltpu.VMEM((CH, hd), jnp.float32) for _ in range(NB)],
            pltpu.VMEM_SHARED((n_pad, hd), jnp.float32),
            [pltpu.SemaphoreType.DMA for _ in range(NB)],
            [pltpu.SemaphoreType.DMA for _ in range(NB)],
        ],
    )
    def agg(h_hbm, src_hbm, dst_hbm, out_hbm, src_v, dst_v, bufs, acc, gsems, ssems):
        c = lax.axis_index("c")
        s = lax.axis_index("s")
        pltpu.sync_copy(src_hbm.at[c, s], src_v)
        pltpu.sync_copy(dst_hbm.at[s], dst_v)

        def zbody(r, carry):
            for k in range(hd // LANES):
                bufs[0][r, pl.ds(k * LANES, LANES)] = jnp.zeros((LANES,), jnp.float32)
            return carry

        lax.fori_loop(0, CH, zbody, 0)
        for blk in range(rows // CH):
            pltpu.sync_copy(bufs[0], acc.at[pl.ds(s * rows + blk * CH, CH)])
        plsc.subcore_barrier()

        for b in range(PF):
            pltpu.async_copy(h_hbm.at[src_v.at[b]], bufs[b], gsems[b])

        def body(i, carry):
            base = i * NB
            for b in range(NB):
                j = base + b
                pltpu.make_async_copy(h_hbm.at[src_v.at[j]], bufs[b], gsems[b]).wait()
                bprev = (b - 1) % NB
                if b == 0:
                    @pl.when(i > 0)
                    def _():
                        pltpu.make_async_copy(
                            bufs[bprev], acc.at[dst_v.at[0]], ssems[bprev]
                        ).wait()
                else:
                    pltpu.make_async_copy(
                        bufs[bprev], acc.at[dst_v.at[0]], ssems[bprev]
                    ).wait()
                pltpu.async_copy(bufs[b], acc.at[dst_v.at[j]], ssems[b], add=True)
                jp = j + PF
                bp = (b + PF) % NB

                @pl.when(jp < kc)
                def _():
                    pltpu.async_copy(h_hbm.at[src_v.at[jp]], bufs[bp], gsems[bp])

            return carry

        lax.fori_loop(0, kc // NB, body, 0)
        pltpu.make_async_copy(bufs[NB - 1], acc.at[dst_v.at[0]], ssems[NB - 1]).wait()
        plsc.subcore_barrier()
        pltpu.sync_copy(
            acc.at[pl.ds(s * rows, rows)],
            out_hbm.at[c, pl.ds(s * rows, rows)],
        )

    return agg


def _prep_tc(x_pad, deg_t):
    def body(x_ref, deg_ref, h_ref):
        norm = lax.rsqrt(jnp.maximum(deg_ref[:, 0:1], 1.0))
        h_ref[...] = x_ref[...] * norm

    return pl.pallas_call(
        body,
        out_shape=jax.ShapeDtypeStruct(x_pad.shape, jnp.float32),
    )(x_pad, deg_t)


def _finish_tc(partials, deg_t, w, b2):
    n_pad = partials.shape[1]

    def body(p_ref, deg_ref, w_ref, b_ref, o_ref):
        p = jnp.concatenate([p_ref[0], p_ref[1]], axis=1)
        norm = lax.rsqrt(jnp.maximum(deg_ref[:, 1:2], 1.0))
        agg = p * norm
        o_ref[...] = (
            jnp.dot(agg, w_ref[...], preferred_element_type=jnp.float32) + b_ref[...]
        )

    return pl.pallas_call(
        body,
        out_shape=jax.ShapeDtypeStruct((n_pad, D), jnp.float32),
    )(partials, deg_t, w, b2)


def kernel(x, edge_index, W, b):
    n, d = x.shape
    assert d == D
    e = edge_index.shape[1]

    n_pad = -(-n // (NS * CH)) * (NS * CH)
    if n_pad == n:
        n_pad += NS * CH
    e_per_t = -(-e // (NS * NB * CH)) * (NB * CH)
    e_pad = e_per_t * NS
    kc = e_per_t // CH

    src = edge_index[0]
    dst = edge_index[1]
    spare = n_pad - n
    fill = (jnp.arange(e_pad - e, dtype=jnp.int32) % spare) + n
    src_p = jnp.concatenate([src, fill])
    dst_p = jnp.concatenate([dst, fill])

    idx2 = jnp.stack([src_p, dst_p]).reshape(NC, NS, kc, CH)
    counts = _deg_call(n_pad, kc)(idx2)
    deg_t = counts.T

    x_pad = jnp.pad(x, ((0, n_pad - n), (0, 0)))
    h = _prep_tc(x_pad, deg_t)
    h2 = h.reshape(2 * n_pad, D // NC)

    srcg = jnp.stack([2 * src_p, 2 * src_p + 1]).reshape(NC, NS, kc, CH)
    partials = _agg_call(n_pad, kc)(h2, srcg, dst_p.reshape(NS, kc, CH))
    out = _finish_tc(partials, deg_t, W, b.reshape(1, D))
    return out[:n]
